# baseline (speedup 1.0000x reference)
import functools

import jax
import jax.numpy as jnp
from jax import lax
from jax.experimental import pallas as pl
from jax.experimental.pallas import tpu as pltpu
from jax.experimental.pallas import tpu_sc as plsc

L = 16
NCORES = 2
NSUB = 16
CB = 32
ROWW = 4 * CB
SBH = 2


def _matmul(X, W):
    N, K = X.shape
    D = W.shape[1]
    BN = 1000

    def body(x_ref, w_ref, o_ref):
        o_ref[...] = jnp.dot(x_ref[...], w_ref[...],
                             preferred_element_type=jnp.float32)

    return pl.pallas_call(
        body,
        grid=(N // BN,),
        in_specs=[pl.BlockSpec((BN, K), lambda i: (i, 0)),
                  pl.BlockSpec((K, D), lambda i: (0, 0))],
        out_specs=pl.BlockSpec((BN, D), lambda i: (i, 0)),
        out_shape=jax.ShapeDtypeStruct((N, D), jnp.float32),
    )(X, W)


def _bias_add(p, b2):
    N, D = p.shape
    BN = 1000

    def body(a_ref, b_ref, o_ref):
        o_ref[...] = a_ref[...] + b_ref[...]

    return pl.pallas_call(
        body,
        grid=(N // BN,),
        in_specs=[pl.BlockSpec((BN, D), lambda i: (i, 0)),
                  pl.BlockSpec((1, D), lambda i: (0, 0))],
        out_specs=pl.BlockSpec((BN, D), lambda i: (i, 0)),
        out_shape=jax.ShapeDtypeStruct((N, D), jnp.float32),
    )(p, b2)


def _spmm_sc(support, src3, dst3, w3):
    N, D = support.shape
    NB, sbh, roww = src3.shape
    assert sbh == SBH and roww == ROWW
    NSB = NB // NSUB
    NT = NSB * 8
    HN = N // NCORES
    DUMP = HN
    SRT = (N // (8 * NSUB)) * 8
    SREM = N - NSUB * SRT
    ART = (HN // (8 * NSUB)) * 8
    AREM = HN - NSUB * ART
    assert D % L == 0 and SREM % 8 == 0 and AREM % 8 == 0
    assert SREM <= CB and AREM + 8 <= CB and HN % 8 == 0

    mesh = plsc.VectorSubcoreMesh(core_axis_name="c", subcore_axis_name="s")

    @functools.partial(
        pl.kernel,
        out_type=jax.ShapeDtypeStruct((NCORES, HN, D), jnp.float32),
        mesh=mesh,
        scratch_types=[
            pltpu.VMEM((CB, D), jnp.float32),
            pltpu.VMEM((CB, D), jnp.float32),
            pltpu.VMEM((SBH, ROWW), jnp.int32),
            pltpu.VMEM((SBH, ROWW), jnp.int32),
            pltpu.VMEM((SBH, ROWW), jnp.int32),
            pltpu.VMEM((SBH, ROWW), jnp.int32),
            pltpu.VMEM((SBH, ROWW), jnp.float32),
            pltpu.VMEM((SBH, ROWW), jnp.float32),
            pltpu.VMEM_SHARED((N, D), jnp.float32),
            pltpu.VMEM_SHARED((HN + 8, D), jnp.float32),
            pltpu.SemaphoreType.DMA,
            pltpu.SemaphoreType.DMA,
            pltpu.SemaphoreType.DMA,
        ],
    )
    def spmm(support_hbm, src_hbm, dst_hbm, w_hbm, out_hbm,
             rows0, rows1, esrc0, esrc1, edst0, edst1, ew0, ew1,
             sup_sp, acc_sh, gsem, ssem, esem):
        c = lax.axis_index("c")
        s = lax.axis_index("s")

        pltpu.sync_copy(support_hbm.at[pl.ds(s * SRT, SRT)],
                        sup_sp.at[pl.ds(s * SRT, SRT)])
        if SREM:
            @pl.when(s == 0)
            def _():
                pltpu.sync_copy(support_hbm.at[pl.ds(NSUB * SRT, SREM)],
                                sup_sp.at[pl.ds(NSUB * SRT, SREM)])

        def zrow(r, carry):
            for dd in range(D // L):
                rows0[r, pl.ds(dd * L, L)] = jnp.zeros((L,), jnp.float32)
            return carry
        lax.fori_loop(0, CB, zrow, 0)
        arow0 = s * ART
        for k in range(ART // CB):
            pltpu.sync_copy(rows0, acc_sh.at[pl.ds(arow0 + k * CB, CB)])
        if ART % CB:
            pltpu.sync_copy(
                rows0.at[pl.ds(0, ART % CB)],
                acc_sh.at[pl.ds(arow0 + (ART // CB) * CB, ART % CB)])
        zrem = (HN + 8) - NSUB * ART
        @pl.when(s == 0)
        def _():
            pltpu.sync_copy(rows0.at[pl.ds(0, zrem)],
                            acc_sh.at[pl.ds(NSUB * ART, zrem)])
        plsc.subcore_barrier()

        rows = (rows0, rows1)
        esrc = (esrc0, esrc1)
        edst = (edst0, edst1)
        ew = (ew0, ew1)
        j0 = s * NSB

        def fetch_block(sb, p, sync):
            if sync:
                pltpu.sync_copy(src_hbm.at[j0 + sb], esrc[p])
                pltpu.sync_copy(dst_hbm.at[j0 + sb], edst[p])
                pltpu.sync_copy(w_hbm.at[j0 + sb], ew[p])
            else:
                pltpu.async_copy(src_hbm.at[j0 + sb], esrc[p], esem)
                pltpu.async_copy(dst_hbm.at[j0 + sb], edst[p], esem)
                pltpu.async_copy(w_hbm.at[j0 + sb], ew[p], esem)

        def wait_block(sb, p):
            pltpu.make_async_copy(src_hbm.at[j0 + sb], esrc[p], esem).wait()
            pltpu.make_async_copy(dst_hbm.at[j0 + sb], edst[p], esem).wait()
            pltpu.make_async_copy(w_hbm.at[j0 + sb], ew[p], esem).wait()

        def rebase_dst(p):
            base = jnp.full((L,), c * HN, jnp.int32)
            dump = jnp.full((L,), DUMP, jnp.int32)
            zero = jnp.zeros((L,), jnp.int32)
            eb = edst[p]
            for v in range(SBH * ROWW // L):
                r, off = v // (ROWW // L), (v % (ROWW // L)) * L
                d = eb[r, pl.ds(off, L)]
                local = d - base
                ok = (local >= zero) & (local < dump)
                eb[r, pl.ds(off, L)] = jnp.where(ok, local, dump)

        def scale(cur, p, k):
            def group(g, carry):
                wv16 = ew[p][k // 4, pl.ds((k % 4) * CB + g * L, L)]
                for ll in range(L):
                    wsp = lax.gather(
                        wv16, jnp.full((L, 1), ll, jnp.int32),
                        lax.GatherDimensionNumbers(
                            offset_dims=(), collapsed_slice_dims=(0,),
                            start_index_map=(0,)),
                        slice_sizes=(1,),
                        mode=lax.GatherScatterMode.PROMISE_IN_BOUNDS)
                    e = g * L + ll
                    for dd in range(D // L):
                        sl = pl.ds(dd * L, L)
                        cur[e, sl] = cur[e, sl] * wsp
                return carry
            lax.fori_loop(0, CB // L, group, 0)

        def src_ref(p, k):
            return esrc[p].at[k // 4, pl.ds((k % 4) * CB, CB)]

        def dst_ref(p, k):
            return edst[p].at[k // 4, pl.ds((k % 4) * CB, CB)]

        fetch_block(0, 0, sync=True)
        pltpu.async_copy(sup_sp.at[src_ref(0, 0)], rows0, gsem)

        def block_body(sb, p):
            @pl.when(sb >= 1)
            def _():
                pltpu.make_async_copy(
                    rows1, acc_sh.at[dst_ref(1 - p, 7)], ssem).wait()

            @pl.when(sb + 1 < NSB)
            def _():
                fetch_block(sb + 1, 1 - p, sync=False)

            rebase_dst(p)

            for k in range(8):
                rp = k % 2
                cur, oth = rows[rp], rows[1 - rp]

                pltpu.make_async_copy(
                    sup_sp.at[src_ref(p, k)], cur, gsem).wait()
                if k > 0:
                    pltpu.make_async_copy(
                        oth, acc_sh.at[dst_ref(p, k - 1)], ssem).wait()
                if k < 7:
                    pltpu.async_copy(
                        sup_sp.at[src_ref(p, k + 1)], oth, gsem)
                else:
                    @pl.when(sb + 1 < NSB)
                    def _():
                        wait_block(sb + 1, 1 - p)
                        pltpu.async_copy(
                            sup_sp.at[src_ref(1 - p, 0)], oth, gsem)
                scale(cur, p, k)
                pltpu.async_copy(
                    cur, acc_sh.at[dst_ref(p, k)], ssem, add=True)

        def block(sb, carry):
            @pl.when(sb % 2 == 0)
            def _():
                block_body(sb, 0)

            @pl.when(sb % 2 == 1)
            def _():
                block_body(sb, 1)
            return carry
        lax.fori_loop(0, NSB, block, 0)

        lastp = (NSB - 1) % 2
        pltpu.make_async_copy(
            rows1, acc_sh.at[dst_ref(lastp, 7)], ssem).wait()
        plsc.subcore_barrier()

        pltpu.sync_copy(acc_sh.at[pl.ds(arow0, ART)],
                        out_hbm.at[c, pl.ds(arow0, ART)])
        if AREM:
            @pl.when(s == 0)
            def _():
                pltpu.sync_copy(acc_sh.at[pl.ds(NSUB * ART, AREM)],
                                out_hbm.at[c, pl.ds(NSUB * ART, AREM)])

    return spmm(support, src3, dst3, w3)


def kernel(X, W, b, edge_index, edge_weight):
    N, _ = X.shape
    D = W.shape[1]
    E = edge_weight.shape[0]
    nrow = -(-E // ROWW)
    rps = -(-nrow // NSUB)
    rps = -(-rps // SBH) * SBH
    e_pad = rps * NSUB * ROWW
    pad = e_pad - E

    def shape3(x):
        return jnp.concatenate(
            [x, jnp.zeros((pad,), x.dtype)]).reshape(-1, SBH, ROWW)

    src = shape3(edge_index[0])
    dst = shape3(edge_index[1])
    ew = shape3(edge_weight)

    support = _matmul(X, W)
    partials = _spmm_sc(support, src, dst, ew)
    return _bias_add(partials.reshape(N, D), b.reshape(1, D))
